# baseline (speedup 1.0000x reference)
import functools

import jax
import jax.numpy as jnp
from jax import lax
from jax.experimental import pallas as pl
from jax.experimental.pallas import tpu as pltpu
from jax.experimental.pallas import tpu_sc as plsc


def _fused_lookup_head(table, idx, wb, n_classes, chunk, gblock_rows):
    B = idx.shape[0]
    V, D = table.shape
    info = plsc.get_sparse_core_info()
    nw = info.num_cores * info.num_subcores
    bpw = B // nw
    nchunks = bpw // chunk
    ngroups = gblock_rows // 16
    mesh = plsc.VectorSubcoreMesh(core_axis_name="c", subcore_axis_name="s")

    @functools.partial(
        pl.kernel,
        mesh=mesh,
        out_type=jax.ShapeDtypeStruct((B * n_classes,), jnp.float32),
        compiler_params=pltpu.CompilerParams(
            use_tc_tiling_on_sc=False, needs_layout_passes=False
        ),
    scratch_types=[
            pltpu.VMEM((chunk,), jnp.int32),
            pltpu.VMEM((chunk, D), jnp.float32),
            pltpu.VMEM((n_classes, chunk), jnp.float32),
            pltpu.VMEM((chunk * n_classes,), jnp.float32),
            pltpu.VMEM((wb.shape[0],), jnp.float32),
            pltpu.SemaphoreType.DMA,
        ],
    )
    def k(tbl_hbm, idx_hbm, wb_hbm, out_hbm, idx_v, rows_v, pair_v, comp_v, wb_v, sem):
        wid = lax.axis_index("s") * info.num_cores + lax.axis_index("c")
        base = wid * bpw
        pltpu.sync_copy(wb_hbm, wb_v)
        iota = lax.iota(jnp.int32, 16)
        tok_per_vec = 16 // n_classes
        c_sel = jnp.bitwise_and(iota, n_classes - 1)
        t_half = jnp.right_shift(iota, n_classes // 2)
        bias = [
            plsc.load_gather(wb_v, [jnp.full((16,), 8 + D * n_classes + c, jnp.int32)])
            for c in range(n_classes)
        ]

        def chunk_body(c, _):
            off = base + c * chunk
            pltpu.sync_copy(idx_hbm.at[pl.ds(off, chunk)], idx_v)
            pltpu.async_copy(tbl_hbm.at[idx_v], rows_v, sem).wait()

            def gblock(gb, _):
                rbase = gb * gblock_rows
                acc = [[bias[c] for c in range(n_classes)] for _ in range(ngroups)]
                rowidx = [rbase + g * 16 + iota for g in range(ngroups)]
                for d in range(D):
                    dcol = jnp.full((16,), d, jnp.int32)
                    w = [
                        plsc.load_gather(
                            wb_v, [jnp.full((16,), 8 + d * n_classes + c, jnp.int32)]
                        )
                        for c in range(n_classes)
                    ]
                    for g in range(ngroups):
                        col = plsc.load_gather(rows_v, [rowidx[g], dcol])
                        for c in range(n_classes):
                            acc[g][c] = acc[g][c] + col * w[c]
                for g in range(ngroups):
                    r0 = rbase + g * 16
                    for cc in range(n_classes):
                        pair_v[cc, pl.ds(r0, 16)] = acc[g][cc]
                    for v in range(n_classes):
                        t_idx = t_half + (r0 + v * tok_per_vec)
                        vec = plsc.load_gather(pair_v, [c_sel, t_idx])
                        comp_v[pl.ds((r0 + v * tok_per_vec) * n_classes, 16)] = vec
                return 0

            lax.fori_loop(0, chunk // gblock_rows, gblock, 0)
            pltpu.sync_copy(
                comp_v, out_hbm.at[pl.ds(off * n_classes, chunk * n_classes)]
            )
            return 0

        lax.fori_loop(0, nchunks, chunk_body, 0)

    return k(table, idx, wb)


def kernel(input_ids, table, W, b):
    batch, seq = input_ids.shape
    D = table.shape[1]
    n_classes = W.shape[1]
    idx = input_ids.reshape(-1).astype(jnp.int32)
    wb = (
        jnp.zeros((128,), jnp.float32)
        .at[8 : 8 + D * n_classes]
        .set(W.reshape(-1))
        .at[8 + D * n_classes : 8 + D * n_classes + n_classes]
        .set(b)
    )
    flat = _fused_lookup_head(
        table, idx, wb, n_classes, chunk=3200, gblock_rows=64
    )
    return flat.reshape(batch, seq, n_classes)


# speedup vs baseline: 1.3068x; 1.3068x over previous
import functools

import jax
import jax.numpy as jnp
from jax import lax
from jax.experimental import pallas as pl
from jax.experimental.pallas import tpu as pltpu
from jax.experimental.pallas import tpu_sc as plsc


def _fused_lookup_head(table, idx, wb, n_classes, chunk, gblock_rows):
    B = idx.shape[0]
    V, D = table.shape
    info = plsc.get_sparse_core_info()
    nw = info.num_cores * info.num_subcores
    bpw = B // nw
    nchunks = bpw // chunk
    ngroups = gblock_rows // 16
    mesh = plsc.VectorSubcoreMesh(core_axis_name="c", subcore_axis_name="s")

    @functools.partial(
        pl.kernel,
        mesh=mesh,
        out_type=jax.ShapeDtypeStruct((B * n_classes,), jnp.float32),
        compiler_params=pltpu.CompilerParams(
            use_tc_tiling_on_sc=False, needs_layout_passes=False
        ),
    scratch_types=[
            pltpu.VMEM((chunk,), jnp.int32),
            pltpu.VMEM((chunk, D), jnp.float32),
            pltpu.VMEM((n_classes, chunk), jnp.float32),
            pltpu.VMEM((chunk * n_classes,), jnp.float32),
            pltpu.VMEM((wb.shape[0],), jnp.float32),
            pltpu.SemaphoreType.DMA,
        ],
    )
    def k(tbl_hbm, idx_hbm, wb_hbm, out_hbm, idx_v, rows_v, pair_v, comp_v, wb_v, sem):
        wid = lax.axis_index("s") * info.num_cores + lax.axis_index("c")
        base = wid * bpw
        pltpu.sync_copy(wb_hbm, wb_v)
        iota = lax.iota(jnp.int32, 16)
        tok_per_vec = 16 // n_classes
        c_sel = jnp.bitwise_and(iota, n_classes - 1)
        t_half = jnp.right_shift(iota, n_classes // 2)
        bias = [
            plsc.load_gather(wb_v, [jnp.full((16,), 8 + D * n_classes + c, jnp.int32)])
            for c in range(n_classes)
        ]

        def chunk_body(c, _):
            off = base + c * chunk
            pltpu.sync_copy(idx_hbm.at[pl.ds(off, chunk)], idx_v)
            pltpu.async_copy(tbl_hbm.at[idx_v], rows_v, sem).wait()

            def gblock(gb, _):
                rbase = gb * gblock_rows
                acc = [[bias[c] for c in range(n_classes)] for _ in range(ngroups)]
                rowidx = [rbase + g * 16 + iota for g in range(ngroups)]
                for k in range(D):
                    d_vec = jnp.bitwise_and(iota + k, D - 1)
                    w = [
                        plsc.load_gather(wb_v, [8 + d_vec * n_classes + c])
                        for c in range(n_classes)
                    ]
                    for g in range(ngroups):
                        col = plsc.load_gather(rows_v, [rowidx[g], d_vec])
                        for c in range(n_classes):
                            acc[g][c] = acc[g][c] + col * w[c]
                for g in range(ngroups):
                    r0 = rbase + g * 16
                    for cc in range(n_classes):
                        pair_v[cc, pl.ds(r0, 16)] = acc[g][cc]
                    for v in range(n_classes):
                        t_idx = t_half + (r0 + v * tok_per_vec)
                        vec = plsc.load_gather(pair_v, [c_sel, t_idx])
                        comp_v[pl.ds((r0 + v * tok_per_vec) * n_classes, 16)] = vec
                return 0

            lax.fori_loop(0, chunk // gblock_rows, gblock, 0)
            pltpu.sync_copy(
                comp_v, out_hbm.at[pl.ds(off * n_classes, chunk * n_classes)]
            )
            return 0

        lax.fori_loop(0, nchunks, chunk_body, 0)

    return k(table, idx, wb)


def kernel(input_ids, table, W, b):
    batch, seq = input_ids.shape
    D = table.shape[1]
    n_classes = W.shape[1]
    idx = input_ids.reshape(-1).astype(jnp.int32)
    wb = (
        jnp.zeros((128,), jnp.float32)
        .at[8 : 8 + D * n_classes]
        .set(W.reshape(-1))
        .at[8 + D * n_classes : 8 + D * n_classes + n_classes]
        .set(b)
    )
    flat = _fused_lookup_head(
        table, idx, wb, n_classes, chunk=3200, gblock_rows=64
    )
    return flat.reshape(batch, seq, n_classes)


# speedup vs baseline: 1.3367x; 1.0229x over previous
import functools

import jax
import jax.numpy as jnp
from jax import lax
from jax.experimental import pallas as pl
from jax.experimental.pallas import tpu as pltpu
from jax.experimental.pallas import tpu_sc as plsc


def _fused_lookup_head(table, idx, wb, n_classes, chunk, gblock_rows):
    B = idx.shape[0]
    V, D = table.shape
    info = plsc.get_sparse_core_info()
    nw = info.num_cores * info.num_subcores
    bpw = B // nw
    nchunks = bpw // chunk
    ngroups = gblock_rows // 16
    mesh = plsc.VectorSubcoreMesh(core_axis_name="c", subcore_axis_name="s")

    @functools.partial(
        pl.kernel,
        mesh=mesh,
        out_type=jax.ShapeDtypeStruct((B * n_classes,), jnp.float32),
        compiler_params=pltpu.CompilerParams(
            use_tc_tiling_on_sc=False, needs_layout_passes=False
        ),
    scratch_types=[
            pltpu.VMEM((chunk,), jnp.int32),
            pltpu.VMEM((chunk,), jnp.int32),
            pltpu.VMEM((chunk, D), jnp.float32),
            pltpu.VMEM((chunk, D), jnp.float32),
            pltpu.VMEM((n_classes, chunk), jnp.float32),
            pltpu.VMEM((chunk * n_classes,), jnp.float32),
            pltpu.VMEM((wb.shape[0],), jnp.float32),
            pltpu.SemaphoreType.DMA,
            pltpu.SemaphoreType.DMA,
        ],
    )
    def k(tbl_hbm, idx_hbm, wb_hbm, out_hbm, idx_v0, idx_v1, rows_v0,
          rows_v1, pair_v, comp_v, wb_v, sem0, sem1):
        wid = lax.axis_index("s") * info.num_cores + lax.axis_index("c")
        base = wid * bpw
        pltpu.sync_copy(wb_hbm, wb_v)
        iota = lax.iota(jnp.int32, 16)
        tok_per_vec = 16 // n_classes
        c_sel = jnp.bitwise_and(iota, n_classes - 1)
        t_half = jnp.right_shift(iota, n_classes // 2)
        bias = [
            plsc.load_gather(wb_v, [jnp.full((16,), 8 + D * n_classes + c, jnp.int32)])
            for c in range(n_classes)
        ]

        idx_bufs = (idx_v0, idx_v1)
        row_bufs = (rows_v0, rows_v1)
        sems = (sem0, sem1)

        def start_fetch(c):
            p = c % 2
            pltpu.sync_copy(idx_hbm.at[pl.ds(base + c * chunk, chunk)], idx_bufs[p])
            return pltpu.async_copy(tbl_hbm.at[idx_bufs[p]], row_bufs[p], sems[p])

        def process_chunk(c, cp):
            rows_v = row_bufs[c % 2]
            off = base + c * chunk
            if c + 1 < nchunks:
                nxt = start_fetch(c + 1)
            else:
                nxt = None
            cp.wait()

            def gblock(gb, _):
                rbase = gb * gblock_rows
                acc = [[bias[c] for c in range(n_classes)] for _ in range(ngroups)]
                rowidx = [rbase + g * 16 + iota for g in range(ngroups)]
                for k in range(D):
                    d_vec = jnp.bitwise_and(iota + k, D - 1)
                    w = [
                        plsc.load_gather(wb_v, [8 + d_vec * n_classes + c])
                        for c in range(n_classes)
                    ]
                    for g in range(ngroups):
                        col = plsc.load_gather(rows_v, [rowidx[g], d_vec])
                        for c in range(n_classes):
                            acc[g][c] = acc[g][c] + col * w[c]
                for g in range(ngroups):
                    r0 = rbase + g * 16
                    for cc in range(n_classes):
                        pair_v[cc, pl.ds(r0, 16)] = acc[g][cc]
                    for v in range(n_classes):
                        t_idx = t_half + (r0 + v * tok_per_vec)
                        vec = plsc.load_gather(pair_v, [c_sel, t_idx])
                        comp_v[pl.ds((r0 + v * tok_per_vec) * n_classes, 16)] = vec
                return 0

            lax.fori_loop(0, chunk // gblock_rows, gblock, 0)
            pltpu.sync_copy(
                comp_v, out_hbm.at[pl.ds(off * n_classes, chunk * n_classes)]
            )
            return nxt

        cp = start_fetch(0)
        for c in range(nchunks):
            cp = process_chunk(c, cp)

    return k(table, idx, wb)


def kernel(input_ids, table, W, b):
    batch, seq = input_ids.shape
    D = table.shape[1]
    n_classes = W.shape[1]
    idx = input_ids.reshape(-1).astype(jnp.int32)
    wb = (
        jnp.zeros((128,), jnp.float32)
        .at[8 : 8 + D * n_classes]
        .set(W.reshape(-1))
        .at[8 + D * n_classes : 8 + D * n_classes + n_classes]
        .set(b)
    )
    flat = _fused_lookup_head(
        table, idx, wb, n_classes, chunk=1600, gblock_rows=64
    )
    return flat.reshape(batch, seq, n_classes)
